# baseline (device time: 11469 ns/iter reference)
import jax
import jax.numpy as jnp
from jax import lax
from jax.experimental import pallas as pl
from jax.experimental.pallas import tpu as pltpu


def kernel(x):
    m_per, n = x.shape
    q = m_per // 4
    h = q // 2

    def body(x_ref, out_ref, xsend_sems, xrecv_sems, ysend_sems, yrecv_sems):
        my_x = lax.axis_index("x")
        my_y = lax.axis_index("y")
        x_nbr = (1 - my_x, my_y)
        y_nbr = (my_x, 1 - my_y)

        barrier_sem = pltpu.get_barrier_semaphore()
        for nbr in (x_nbr, y_nbr):
            pl.semaphore_signal(
                barrier_sem, inc=1, device_id=nbr,
                device_id_type=pl.DeviceIdType.MESH,
            )
        pl.semaphore_wait(barrier_sem, 2)

        out_ref[pl.ds(my_x * m_per, m_per), :] = x_ref[...].astype(jnp.bfloat16)

        mine = my_x * m_per
        theirs = (1 - my_x) * m_per

        r_out = 3 * my_y * q
        plan = [(r_out, h), (r_out + h, h), (q, q), (2 * q, q)]

        x_rdmas = []
        for i, (off, rows) in enumerate(plan):
            rdma = pltpu.make_async_remote_copy(
                src_ref=out_ref.at[pl.ds(mine + off, rows), :],
                dst_ref=out_ref.at[pl.ds(mine + off, rows), :],
                send_sem=xsend_sems.at[i],
                recv_sem=xrecv_sems.at[i],
                device_id=x_nbr,
                device_id_type=pl.DeviceIdType.MESH,
            )
            rdma.start()
            x_rdmas.append(rdma)

        y_rdmas = []
        for i in range(2):
            off = r_out + i * h
            x_rdmas[i].wait_recv()
            rdma = pltpu.make_async_remote_copy(
                src_ref=out_ref.at[pl.ds(theirs + off, h), :],
                dst_ref=out_ref.at[pl.ds(theirs + off, h), :],
                send_sem=ysend_sems.at[i],
                recv_sem=yrecv_sems.at[i],
                device_id=y_nbr,
                device_id_type=pl.DeviceIdType.MESH,
            )
            rdma.start()
            y_rdmas.append(rdma)

        for i in range(2, 4):
            x_rdmas[i].wait_recv()
        for r in y_rdmas:
            r.wait_recv()
        for r in x_rdmas:
            r.wait_send()
        for r in y_rdmas:
            r.wait_send()

    return pl.pallas_call(
        body,
        out_shape=jax.ShapeDtypeStruct((2 * m_per, n), jnp.bfloat16),
        in_specs=[pl.BlockSpec(memory_space=pltpu.VMEM)],
        out_specs=pl.BlockSpec(memory_space=pltpu.VMEM),
        scratch_shapes=[
            pltpu.SemaphoreType.DMA((4,)),
            pltpu.SemaphoreType.DMA((4,)),
            pltpu.SemaphoreType.DMA((2,)),
            pltpu.SemaphoreType.DMA((2,)),
        ],
        compiler_params=pltpu.CompilerParams(collective_id=0),
    )(x)
